# Initial kernel scaffold; baseline (speedup 1.0000x reference)
import jax
import jax.numpy as jnp
from jax.experimental import pallas as pl


def kernel(h, edge_index, Ws, bs, gammas, betas, pW1, pb1, pW2, pb2):
    raise NotImplementedError("write your pallas kernel here")


# speedup vs baseline: 5.2562x; 5.2562x over previous
import functools

import jax
import jax.numpy as jnp
from jax import lax
from jax.experimental import pallas as pl
from jax.experimental.pallas import tpu as pltpu
from jax.experimental.pallas import tpu_sc as plsc

NNODES = 10000
D = 128
NEDGE = 320000
NCLS = 2
NLAYER = 4

NPAD = 10240
NSUB = 16
NCORE = 2
NW = NSUB * NCORE
CHUNK = 128
STEPS = 80
EPT = STEPS * CHUNK
EPAD = NW * EPT
RPT = NPAD // NSUB

_f32 = jnp.float32


def _mesh():
    return plsc.VectorSubcoreMesh(
        core_axis_name="c", subcore_axis_name="s",
        num_cores=NCORE, num_subcores=NSUB)


@functools.partial(
    pl.kernel,
    out_type=jax.ShapeDtypeStruct((NCORE * NPAD, D), _f32),
    mesh=_mesh(),
    scratch_types=[
        pltpu.VMEM((STEPS, CHUNK), jnp.int32),
        pltpu.VMEM((STEPS, CHUNK), jnp.int32),
        pltpu.VMEM((CHUNK, D), _f32),
        pltpu.VMEM_SHARED((NPAD, D), _f32),
        pltpu.SemaphoreType.DMA,
    ],
)
def _sc_spmm(hn, srcm, dstm, z128, out, src_v, dst_v, msg, acc, sem):
    c = lax.axis_index("c")
    s = lax.axis_index("s")
    wid = c * NSUB + s
    base = s * RPT
    pltpu.sync_copy(z128, msg)
    for k in range(RPT // CHUNK):
        pltpu.sync_copy(msg, acc.at[pl.ds(base + k * CHUNK, CHUNK)])
    pltpu.sync_copy(srcm.at[pl.ds(wid * STEPS, STEPS)], src_v)
    pltpu.sync_copy(dstm.at[pl.ds(wid * STEPS, STEPS)], dst_v)
    plsc.subcore_barrier()

    def body(j, carry):
        pltpu.async_copy(hn.at[src_v.at[j]], msg, sem).wait()
        pltpu.sync_copy(msg, acc.at[dst_v.at[j]], add=True)
        return carry

    lax.fori_loop(0, STEPS, body, 0)
    plsc.subcore_barrier()
    for k in range(RPT // CHUNK):
        sl = pl.ds(base + k * CHUNK, CHUNK)
        pltpu.sync_copy(acc.at[sl], msg)
        pltpu.sync_copy(msg, out.at[pl.ds(c * NPAD + base + k * CHUNK, CHUNK)])


@functools.partial(
    pl.kernel,
    out_type=jax.ShapeDtypeStruct((EPAD, D), _f32),
    mesh=_mesh(),
    scratch_types=[
        pltpu.VMEM((STEPS, CHUNK), jnp.int32),
        pltpu.VMEM((STEPS, CHUNK), jnp.int32),
        pltpu.VMEM((CHUNK, D), _f32),
        pltpu.VMEM((CHUNK, D), _f32),
        pltpu.SemaphoreType.DMA,
        pltpu.SemaphoreType.DMA,
    ],
)
def _sc_edge(u, v, srcm, dstm, out, src_v, dst_v, bufa, bufb, sema, semb):
    c = lax.axis_index("c")
    s = lax.axis_index("s")
    wid = c * NSUB + s
    pltpu.sync_copy(srcm.at[pl.ds(wid * STEPS, STEPS)], src_v)
    pltpu.sync_copy(dstm.at[pl.ds(wid * STEPS, STEPS)], dst_v)

    def body(j, carry):
        ca = pltpu.async_copy(u.at[src_v.at[j]], bufa, sema)
        cb = pltpu.async_copy(v.at[dst_v.at[j]], bufb, semb)
        ca.wait()
        cb.wait()

        def row(r, c2):
            for kk in range(D // 16):
                sl = pl.ds(kk * 16, 16)
                bufa[r, sl] = bufa[r, sl] + bufb[r, sl]
            return c2

        lax.fori_loop(0, CHUNK, row, 0)
        pltpu.sync_copy(bufa, out.at[pl.ds(wid * EPT + j * CHUNK, CHUNK)])
        return carry

    lax.fori_loop(0, STEPS, body, 0)


def _norm_body(do_ref, di_ref, o_ref):
    to = do_ref[0] + do_ref[1]
    ti = di_ref[0] + di_ref[1]
    o_ref[0:1, :] = lax.rsqrt(jnp.maximum(to[:, 0:1], 1.0)).T
    o_ref[1:2, :] = lax.rsqrt(jnp.maximum(ti[:, 0:1], 1.0)).T


_tc_norms = pl.pallas_call(
    _norm_body,
    grid=(NPAD // 1024,),
    in_specs=[
        pl.BlockSpec((NCORE, 1024, D), lambda i: (0, i, 0)),
        pl.BlockSpec((NCORE, 1024, D), lambda i: (0, i, 0)),
    ],
    out_specs=pl.BlockSpec((2, 1024), lambda i: (0, i)),
    out_shape=jax.ShapeDtypeStruct((2, NPAD), _f32),
)


def _mm_body(h_ref, w_ref, b_ref, n_ref, o_ref):
    hw = jnp.dot(h_ref[...], w_ref[...], preferred_element_type=_f32)
    o_ref[...] = (hw + b_ref[...]) * n_ref[0][:, None]


_tc_mm = pl.pallas_call(
    _mm_body,
    grid=(NPAD // 256,),
    in_specs=[
        pl.BlockSpec((256, D), lambda i: (i, 0)),
        pl.BlockSpec((D, D), lambda i: (0, 0)),
        pl.BlockSpec((1, D), lambda i: (0, 0)),
        pl.BlockSpec((2, 256), lambda i: (0, i)),
    ],
    out_specs=pl.BlockSpec((256, D), lambda i: (i, 0)),
    out_shape=jax.ShapeDtypeStruct((NPAD, D), _f32),
)


def _bn_body(p_ref, n_ref, g_ref, be_ref, h_ref, o_ref, acc_ref):
    ph = pl.program_id(0)
    i = pl.program_id(1)
    pre = (p_ref[0] + p_ref[1]) * n_ref[1][:, None]
    rows = lax.broadcasted_iota(jnp.int32, (256, 1), 0) + i * 256
    msk = rows < NNODES

    @pl.when(ph == 0)
    def _():
        @pl.when(i == 0)
        def _():
            acc_ref[...] = jnp.zeros_like(acc_ref)
        pm = jnp.where(msk, pre, 0.0)
        acc_ref[0:1, :] += jnp.sum(pm, axis=0, keepdims=True)
        acc_ref[1:2, :] += jnp.sum(pm * pm, axis=0, keepdims=True)
        o_ref[...] = pre

    @pl.when(ph == 1)
    def _():
        mean = acc_ref[0:1, :] * (1.0 / NNODES)
        var = acc_ref[1:2, :] * (1.0 / NNODES) - mean * mean
        inv = lax.rsqrt(var + 1e-5)
        y = (pre - mean) * inv * g_ref[...] + be_ref[...]
        o_ref[...] = jnp.maximum(y, 0.0) + h_ref[...]


_tc_bn = pl.pallas_call(
    _bn_body,
    grid=(2, NPAD // 256),
    in_specs=[
        pl.BlockSpec((NCORE, 256, D), lambda p, i: (0, i, 0)),
        pl.BlockSpec((2, 256), lambda p, i: (0, i)),
        pl.BlockSpec((1, D), lambda p, i: (0, 0)),
        pl.BlockSpec((1, D), lambda p, i: (0, 0)),
        pl.BlockSpec((256, D), lambda p, i: (i, 0)),
    ],
    out_specs=pl.BlockSpec((256, D), lambda p, i: (i, 0)),
    out_shape=jax.ShapeDtypeStruct((NPAD, D), _f32),
    scratch_shapes=[pltpu.VMEM((8, 128), _f32)],
)


def _uv_body(h_ref, w_ref, u_ref, v_ref):
    hh = h_ref[...]
    w = w_ref[...]
    u_ref[...] = jnp.dot(hh, w[:D], preferred_element_type=_f32)
    v_ref[...] = jnp.dot(hh, w[D:], preferred_element_type=_f32)


_tc_uv = pl.pallas_call(
    _uv_body,
    grid=(NPAD // 256,),
    in_specs=[
        pl.BlockSpec((256, D), lambda i: (i, 0)),
        pl.BlockSpec((2 * D, D), lambda i: (0, 0)),
    ],
    out_specs=[
        pl.BlockSpec((256, D), lambda i: (i, 0)),
        pl.BlockSpec((256, D), lambda i: (i, 0)),
    ],
    out_shape=[
        jax.ShapeDtypeStruct((NPAD, D), _f32),
        jax.ShapeDtypeStruct((NPAD, D), _f32),
    ],
)


def _scores_body(s_ref, pb1_ref, w2_ref, pb2_ref, o_ref):
    r = jnp.maximum(s_ref[...] + pb1_ref[...], 0.0)
    res = jnp.dot(r, w2_ref[...], preferred_element_type=_f32) + pb2_ref[...]
    o_ref[...] = res[:, :NCLS].T


_tc_scores = pl.pallas_call(
    _scores_body,
    grid=(EPAD // 1024,),
    in_specs=[
        pl.BlockSpec((1024, D), lambda i: (i, 0)),
        pl.BlockSpec((1, D), lambda i: (0, 0)),
        pl.BlockSpec((D, 128), lambda i: (0, 0)),
        pl.BlockSpec((1, 128), lambda i: (0, 0)),
    ],
    out_specs=pl.BlockSpec((2, 1024), lambda i: (0, i)),
    out_shape=jax.ShapeDtypeStruct((2, EPAD), _f32),
)


def kernel(h, edge_index, Ws, bs, gammas, betas, pW1, pb1, pW2, pb2):
    src = edge_index[0].astype(jnp.int32)
    dst = edge_index[1].astype(jnp.int32)
    pad = NNODES + (jnp.arange(EPAD - NEDGE, dtype=jnp.int32) % (NPAD - NNODES))
    srcm = jnp.concatenate([src, pad]).reshape(NW * STEPS, CHUNK)
    dstm = jnp.concatenate([dst, pad]).reshape(NW * STEPS, CHUNK)
    hp = jnp.zeros((NPAD, D), _f32).at[:NNODES].set(h)

    z128 = jnp.zeros((CHUNK, D), _f32)
    e0 = jnp.zeros((NPAD, D), _f32).at[:, 0].set(1.0)

    pout = _sc_spmm(e0, dstm, srcm, z128).reshape(NCORE, NPAD, D)
    pin = _sc_spmm(e0, srcm, dstm, z128).reshape(NCORE, NPAD, D)
    norms = _tc_norms(pout, pin)

    x = hp
    for l in range(NLAYER):
        hn = _tc_mm(x, Ws[l], bs[l].reshape(1, D), norms)
        parts = _sc_spmm(hn, srcm, dstm, z128).reshape(NCORE, NPAD, D)
        x = _tc_bn(parts, norms, gammas[l].reshape(1, D), betas[l].reshape(1, D), x)

    u, v = _tc_uv(x, pW1)
    s = _sc_edge(u, v, srcm, dstm)
    pb2p = jnp.zeros((1, 128), _f32).at[0, :NCLS].set(pb2)
    pw2p = jnp.zeros((D, 128), _f32).at[:, :NCLS].set(pW2)
    sct = _tc_scores(s, pb1.reshape(1, D), pw2p, pb2p)
    return sct[:, :NEDGE].T


# speedup vs baseline: 7.4909x; 1.4251x over previous
import functools

import jax
import jax.numpy as jnp
from jax import lax
from jax.experimental import pallas as pl
from jax.experimental.pallas import tpu as pltpu
from jax.experimental.pallas import tpu_sc as plsc

NNODES = 10000
D = 128
NEDGE = 320000
NCLS = 2
NLAYER = 4

NPAD = 10240
NSUB = 16
NCORE = 2
NW = NSUB * NCORE
CHUNK = 128
STEPS = 80
EPT = STEPS * CHUNK
EPAD = NW * EPT
RPT = NPAD // NSUB

_f32 = jnp.float32


def _mesh():
    return plsc.VectorSubcoreMesh(
        core_axis_name="c", subcore_axis_name="s",
        num_cores=NCORE, num_subcores=NSUB)


@functools.partial(
    pl.kernel,
    out_type=jax.ShapeDtypeStruct((NCORE * NPAD, D), _f32),
    mesh=_mesh(),
    scratch_types=[
        pltpu.VMEM((STEPS // 2, CHUNK), jnp.int32),
        pltpu.VMEM((STEPS // 2, CHUNK), jnp.int32),
        pltpu.VMEM((CHUNK, D), _f32),
        pltpu.VMEM((CHUNK, D), _f32),
        pltpu.VMEM_SHARED((NPAD, D), _f32),
        pltpu.SemaphoreType.DMA,
        pltpu.SemaphoreType.DMA,
    ],
)
def _sc_spmm(hn, srcm, dstm, z128, out, src_v, dst_v, msg0, msg1, acc, sem0, sem1):
    c = lax.axis_index("c")
    s = lax.axis_index("s")
    wid = c * NSUB + s
    base = s * RPT
    hsteps = STEPS // 2
    pltpu.sync_copy(z128, msg0)
    for k in range(RPT // CHUNK):
        pltpu.sync_copy(msg0, acc.at[pl.ds(base + k * CHUNK, CHUNK)])
    plsc.subcore_barrier()

    for half in range(2):
        off = wid * STEPS + half * hsteps
        pltpu.sync_copy(srcm.at[pl.ds(off, hsteps)], src_v)
        pltpu.sync_copy(dstm.at[pl.ds(off, hsteps)], dst_v)
        pltpu.async_copy(hn.at[src_v.at[0]], msg0, sem0)

        def body(i, carry):
            j0 = 2 * i
            pltpu.async_copy(hn.at[src_v.at[j0 + 1]], msg1, sem1)
            pltpu.make_async_copy(hn.at[src_v.at[j0]], msg0, sem0).wait()
            pltpu.sync_copy(msg0, acc.at[dst_v.at[j0]], add=True)
            pltpu.async_copy(hn.at[src_v.at[j0 + 2]], msg0, sem0)
            pltpu.make_async_copy(hn.at[src_v.at[j0 + 1]], msg1, sem1).wait()
            pltpu.sync_copy(msg1, acc.at[dst_v.at[j0 + 1]], add=True)
            return carry

        lax.fori_loop(0, hsteps // 2 - 1, body, 0)
        pltpu.async_copy(hn.at[src_v.at[hsteps - 1]], msg1, sem1)
        pltpu.make_async_copy(hn.at[src_v.at[hsteps - 2]], msg0, sem0).wait()
        pltpu.sync_copy(msg0, acc.at[dst_v.at[hsteps - 2]], add=True)
        pltpu.make_async_copy(hn.at[src_v.at[hsteps - 1]], msg1, sem1).wait()
        pltpu.sync_copy(msg1, acc.at[dst_v.at[hsteps - 1]], add=True)
    plsc.subcore_barrier()
    for k in range(RPT // CHUNK):
        sl = pl.ds(base + k * CHUNK, CHUNK)
        pltpu.sync_copy(acc.at[sl], msg0)
        pltpu.sync_copy(msg0, out.at[pl.ds(c * NPAD + base + k * CHUNK, CHUNK)])


@functools.partial(
    pl.kernel,
    out_type=jax.ShapeDtypeStruct((EPAD, D), _f32),
    mesh=_mesh(),
    scratch_types=[
        pltpu.VMEM((STEPS, CHUNK), jnp.int32),
        pltpu.VMEM((STEPS, CHUNK), jnp.int32),
        pltpu.VMEM((CHUNK, D), _f32),
        pltpu.VMEM((CHUNK, D), _f32),
        pltpu.VMEM((CHUNK, D), _f32),
        pltpu.VMEM((CHUNK, D), _f32),
        pltpu.SemaphoreType.DMA,
        pltpu.SemaphoreType.DMA,
        pltpu.SemaphoreType.DMA,
        pltpu.SemaphoreType.DMA,
    ],
)
def _sc_edge(u, v, srcm, dstm, out,
             src_v, dst_v, bufa0, bufb0, bufa1, bufb1, sa0, sb0, sa1, sb1):
    c = lax.axis_index("c")
    s = lax.axis_index("s")
    wid = c * NSUB + s
    pltpu.sync_copy(srcm.at[pl.ds(wid * STEPS, STEPS)], src_v)
    pltpu.sync_copy(dstm.at[pl.ds(wid * STEPS, STEPS)], dst_v)

    def start(j, ba, bb, sa, sb):
        pltpu.async_copy(u.at[src_v.at[j]], ba, sa)
        pltpu.async_copy(v.at[dst_v.at[j]], bb, sb)

    def finish(j, ba, bb, sa, sb):
        pltpu.make_async_copy(u.at[src_v.at[j]], ba, sa).wait()
        pltpu.make_async_copy(v.at[dst_v.at[j]], bb, sb).wait()

        def row(r, c2):
            for kk in range(D // 16):
                sl = pl.ds(kk * 16, 16)
                ba[r, sl] = ba[r, sl] + bb[r, sl]
            return c2

        lax.fori_loop(0, CHUNK, row, 0)
        pltpu.sync_copy(ba, out.at[pl.ds(wid * EPT + j * CHUNK, CHUNK)])

    start(0, bufa0, bufb0, sa0, sb0)

    def body(i, carry):
        j0 = 2 * i
        start(j0 + 1, bufa1, bufb1, sa1, sb1)
        finish(j0, bufa0, bufb0, sa0, sb0)
        start(j0 + 2, bufa0, bufb0, sa0, sb0)
        finish(j0 + 1, bufa1, bufb1, sa1, sb1)
        return carry

    lax.fori_loop(0, STEPS // 2 - 1, body, 0)
    start(STEPS - 1, bufa1, bufb1, sa1, sb1)
    finish(STEPS - 2, bufa0, bufb0, sa0, sb0)
    finish(STEPS - 1, bufa1, bufb1, sa1, sb1)


def _norm_body(do_ref, di_ref, o_ref):
    to = do_ref[0] + do_ref[1]
    ti = di_ref[0] + di_ref[1]
    o_ref[0:1, :] = lax.rsqrt(jnp.maximum(to[:, 0:1], 1.0)).T
    o_ref[1:2, :] = lax.rsqrt(jnp.maximum(ti[:, 0:1], 1.0)).T


_tc_norms = pl.pallas_call(
    _norm_body,
    grid=(NPAD // 1024,),
    in_specs=[
        pl.BlockSpec((NCORE, 1024, D), lambda i: (0, i, 0)),
        pl.BlockSpec((NCORE, 1024, D), lambda i: (0, i, 0)),
    ],
    out_specs=pl.BlockSpec((2, 1024), lambda i: (0, i)),
    out_shape=jax.ShapeDtypeStruct((2, NPAD), _f32),
)


def _mm_body(h_ref, w_ref, b_ref, n_ref, o_ref):
    hw = jnp.dot(h_ref[...], w_ref[...], preferred_element_type=_f32)
    o_ref[...] = (hw + b_ref[...]) * n_ref[0][:, None]


_tc_mm = pl.pallas_call(
    _mm_body,
    grid=(NPAD // 256,),
    in_specs=[
        pl.BlockSpec((256, D), lambda i: (i, 0)),
        pl.BlockSpec((D, D), lambda i: (0, 0)),
        pl.BlockSpec((1, D), lambda i: (0, 0)),
        pl.BlockSpec((2, 256), lambda i: (0, i)),
    ],
    out_specs=pl.BlockSpec((256, D), lambda i: (i, 0)),
    out_shape=jax.ShapeDtypeStruct((NPAD, D), _f32),
)


def _bn_core(p_ref, n_ref, g_ref, be_ref, h_ref, acc_ref, ph, i):
    pre = (p_ref[0] + p_ref[1]) * n_ref[1][:, None]
    rows = lax.broadcasted_iota(jnp.int32, (256, 1), 0) + i * 256
    msk = rows < NNODES

    @pl.when(ph == 0)
    def _():
        @pl.when(i == 0)
        def _():
            acc_ref[...] = jnp.zeros_like(acc_ref)
        pm = jnp.where(msk, pre, 0.0)
        acc_ref[0:1, :] += jnp.sum(pm, axis=0, keepdims=True)
        acc_ref[1:2, :] += jnp.sum(pm * pm, axis=0, keepdims=True)

    mean = acc_ref[0:1, :] * (1.0 / NNODES)
    var = acc_ref[1:2, :] * (1.0 / NNODES) - mean * mean
    inv = lax.rsqrt(var + 1e-5)
    y = (pre - mean) * inv * g_ref[...] + be_ref[...]
    return jnp.maximum(y, 0.0) + h_ref[...]


def _bn_mm_body(p_ref, n_ref, g_ref, be_ref, h_ref, w_ref, b_ref,
                xn_ref, hn_ref, acc_ref):
    ph = pl.program_id(0)
    i = pl.program_id(1)
    y = _bn_core(p_ref, n_ref, g_ref, be_ref, h_ref, acc_ref, ph, i)
    xn_ref[...] = y
    hn_ref[...] = (jnp.dot(y, w_ref[...], preferred_element_type=_f32)
                   + b_ref[...]) * n_ref[0][:, None]


_tc_bn_mm = pl.pallas_call(
    _bn_mm_body,
    grid=(2, NPAD // 256),
    in_specs=[
        pl.BlockSpec((NCORE, 256, D), lambda p, i: (0, i, 0)),
        pl.BlockSpec((2, 256), lambda p, i: (0, i)),
        pl.BlockSpec((1, D), lambda p, i: (0, 0)),
        pl.BlockSpec((1, D), lambda p, i: (0, 0)),
        pl.BlockSpec((256, D), lambda p, i: (i, 0)),
        pl.BlockSpec((D, D), lambda p, i: (0, 0)),
        pl.BlockSpec((1, D), lambda p, i: (0, 0)),
    ],
    out_specs=[
        pl.BlockSpec((256, D), lambda p, i: (i, 0)),
        pl.BlockSpec((256, D), lambda p, i: (i, 0)),
    ],
    out_shape=[
        jax.ShapeDtypeStruct((NPAD, D), _f32),
        jax.ShapeDtypeStruct((NPAD, D), _f32),
    ],
    scratch_shapes=[pltpu.VMEM((8, 128), _f32)],
)


def _bn_uv_body(p_ref, n_ref, g_ref, be_ref, h_ref, w_ref, u_ref, v_ref, acc_ref):
    ph = pl.program_id(0)
    i = pl.program_id(1)
    y = _bn_core(p_ref, n_ref, g_ref, be_ref, h_ref, acc_ref, ph, i)
    w = w_ref[...]
    u_ref[...] = jnp.dot(y, w[:D], preferred_element_type=_f32)
    v_ref[...] = jnp.dot(y, w[D:], preferred_element_type=_f32)


_tc_bn_uv = pl.pallas_call(
    _bn_uv_body,
    grid=(2, NPAD // 256),
    in_specs=[
        pl.BlockSpec((NCORE, 256, D), lambda p, i: (0, i, 0)),
        pl.BlockSpec((2, 256), lambda p, i: (0, i)),
        pl.BlockSpec((1, D), lambda p, i: (0, 0)),
        pl.BlockSpec((1, D), lambda p, i: (0, 0)),
        pl.BlockSpec((256, D), lambda p, i: (i, 0)),
        pl.BlockSpec((2 * D, D), lambda p, i: (0, 0)),
    ],
    out_specs=[
        pl.BlockSpec((256, D), lambda p, i: (i, 0)),
        pl.BlockSpec((256, D), lambda p, i: (i, 0)),
    ],
    out_shape=[
        jax.ShapeDtypeStruct((NPAD, D), _f32),
        jax.ShapeDtypeStruct((NPAD, D), _f32),
    ],
    scratch_shapes=[pltpu.VMEM((8, 128), _f32)],
)


def _scores_body(s_ref, pb1_ref, w2_ref, pb2_ref, o_ref):
    r = jnp.maximum(s_ref[...] + pb1_ref[...], 0.0)
    res = jnp.dot(r, w2_ref[...], preferred_element_type=_f32) + pb2_ref[...]
    o_ref[...] = res[:, :NCLS].T


_tc_scores = pl.pallas_call(
    _scores_body,
    grid=(EPAD // 1024,),
    in_specs=[
        pl.BlockSpec((1024, D), lambda i: (i, 0)),
        pl.BlockSpec((1, D), lambda i: (0, 0)),
        pl.BlockSpec((D, 128), lambda i: (0, 0)),
        pl.BlockSpec((1, 128), lambda i: (0, 0)),
    ],
    out_specs=pl.BlockSpec((2, 1024), lambda i: (0, i)),
    out_shape=jax.ShapeDtypeStruct((2, EPAD), _f32),
)


def kernel(h, edge_index, Ws, bs, gammas, betas, pW1, pb1, pW2, pb2):
    src = edge_index[0].astype(jnp.int32)
    dst = edge_index[1].astype(jnp.int32)
    pad = NNODES + (jnp.arange(EPAD - NEDGE, dtype=jnp.int32) % (NPAD - NNODES))
    srcm = jnp.concatenate([src, pad]).reshape(NW * STEPS, CHUNK)
    dstm = jnp.concatenate([dst, pad]).reshape(NW * STEPS, CHUNK)
    hp = jnp.zeros((NPAD, D), _f32).at[:NNODES].set(h)

    z128 = jnp.zeros((CHUNK, D), _f32)
    e0 = jnp.zeros((NPAD, D), _f32).at[:, 0].set(1.0)

    pout = _sc_spmm(e0, dstm, srcm, z128).reshape(NCORE, NPAD, D)
    pin = _sc_spmm(e0, srcm, dstm, z128).reshape(NCORE, NPAD, D)
    norms = _tc_norms(pout, pin)

    x = hp
    hn = _tc_mm(x, Ws[0], bs[0].reshape(1, D), norms)
    for l in range(NLAYER):
        parts = _sc_spmm(hn, srcm, dstm, z128).reshape(NCORE, NPAD, D)
        g = gammas[l].reshape(1, D)
        be = betas[l].reshape(1, D)
        if l < NLAYER - 1:
            x, hn = _tc_bn_mm(parts, norms, g, be, x,
                              Ws[l + 1], bs[l + 1].reshape(1, D))
        else:
            u, v = _tc_bn_uv(parts, norms, g, be, x, pW1)

    s = _sc_edge(u, v, srcm, dstm)
    pb2p = jnp.zeros((1, 128), _f32).at[0, :NCLS].set(pb2)
    pw2p = jnp.zeros((D, 128), _f32).at[:, :NCLS].set(pW2)
    sct = _tc_scores(s, pb1.reshape(1, D), pw2p, pb2p)
    return sct[:, :NEDGE].T
